# Initial kernel scaffold; baseline (speedup 1.0000x reference)
import jax
import jax.numpy as jnp
from jax.experimental import pallas as pl


def kernel(inputs, pos_embed_weights):
    raise NotImplementedError("write your pallas kernel here")


# speedup vs baseline: 2.2266x; 2.2266x over previous
import functools

import jax
import jax.numpy as jnp
from jax import lax
from jax.experimental import pallas as pl
from jax.experimental.pallas import tpu as pltpu
from jax.experimental.pallas import tpu_sc as plsc

_NUM_POS = 8192
_EMBED_DIM = 768
_B = 4
_S = 8192
_TOTAL = _B * _S

_NC = 2
_NS = 16
_NW = _NC * _NS
_ROWS_PER_W = _TOTAL // _NW
_CHUNK = 64
_N_CHUNKS = _ROWS_PER_W // _CHUNK

_mesh = plsc.VectorSubcoreMesh(core_axis_name="c", subcore_axis_name="s")


@functools.partial(
    pl.kernel,
    mesh=_mesh,
    out_type=jax.ShapeDtypeStruct((_TOTAL, _EMBED_DIM), jnp.float32),
    scratch_types=[
        pltpu.VMEM((_N_CHUNKS, _CHUNK), jnp.int32),
        pltpu.VMEM((2, _CHUNK, _EMBED_DIM), jnp.float32),
        pltpu.SemaphoreType.DMA,
    ],
)
def _gather_rows(idx_hbm, table_hbm, out_hbm, idx_v, rows_v, sem):
    wid = lax.axis_index("s") * _NC + lax.axis_index("c")
    base = wid * _ROWS_PER_W
    pltpu.sync_copy(idx_hbm.at[wid], idx_v)

    copies = [None, None]
    copies[0] = pltpu.async_copy(table_hbm.at[idx_v.at[0]], rows_v.at[0], sem)
    for i in range(_N_CHUNKS):
        buf = i % 2
        nxt = (i + 1) % 2
        if i + 1 < _N_CHUNKS:
            copies[nxt] = pltpu.async_copy(
                table_hbm.at[idx_v.at[i + 1]], rows_v.at[nxt], sem)
        copies[buf].wait()
        pltpu.sync_copy(rows_v.at[buf],
                        out_hbm.at[pl.ds(base + i * _CHUNK, _CHUNK)])


def kernel(inputs, pos_embed_weights):
    idx = inputs.astype(jnp.int32).reshape(_NW, _N_CHUNKS, _CHUNK)
    out = _gather_rows(idx, pos_embed_weights)
    return out.reshape(_B, _S, _EMBED_DIM), pos_embed_weights


# speedup vs baseline: 2.2734x; 1.0210x over previous
import functools

import jax
import jax.numpy as jnp
from jax import lax
from jax.experimental import pallas as pl
from jax.experimental.pallas import tpu as pltpu
from jax.experimental.pallas import tpu_sc as plsc

_NUM_POS = 8192
_EMBED_DIM = 768
_B = 4
_S = 8192
_TOTAL = _B * _S

_NC = 2
_NS = 16
_NW = _NC * _NS
_ROWS_PER_W = _TOTAL // _NW
_CHUNK = 32
_N_CHUNKS = _ROWS_PER_W // _CHUNK
_NBUF = 4
_N_GROUPS = _N_CHUNKS // _NBUF

_mesh = plsc.VectorSubcoreMesh(core_axis_name="c", subcore_axis_name="s")


@functools.partial(
    pl.kernel,
    mesh=_mesh,
    out_type=jax.ShapeDtypeStruct((_TOTAL, _EMBED_DIM), jnp.float32),
    scratch_types=[
        pltpu.VMEM((_N_CHUNKS, _CHUNK), jnp.int32),
        pltpu.VMEM((_NBUF, _CHUNK, _EMBED_DIM), jnp.float32),
        pltpu.SemaphoreType.DMA,
        pltpu.SemaphoreType.DMA,
    ],
)
def _gather_rows(idx_hbm, table_hbm, out_hbm, idx_v, rows_v, sg, sw):
    wid = lax.axis_index("s") * _NC + lax.axis_index("c")
    base = wid * _ROWS_PER_W
    pltpu.sync_copy(idx_hbm.at[wid], idx_v)

    for b in range(_NBUF):
        pltpu.async_copy(table_hbm.at[idx_v.at[b]], rows_v.at[b], sg)

    def _wait_gather(b):
        pltpu.make_async_copy(
            table_hbm.at[idx_v.at[0]], rows_v.at[b], sg).wait()

    def _write_out(i, b):
        dst = out_hbm.at[pl.ds(base + i * _CHUNK, _CHUNK)]
        pltpu.async_copy(rows_v.at[b], dst, sw)
        return dst

    def _group(g, carry):
        for b in range(_NBUF):
            i = g * _NBUF + b
            _wait_gather(b)
            dst = _write_out(i, b)
            pltpu.make_async_copy(rows_v.at[b], dst, sw).wait()
            pltpu.async_copy(
                table_hbm.at[idx_v.at[i + _NBUF]], rows_v.at[b], sg)
        return carry

    lax.fori_loop(0, _N_GROUPS - 1, _group, 0)

    dsts = []
    for b in range(_NBUF):
        i = (_N_GROUPS - 1) * _NBUF + b
        _wait_gather(b)
        dsts.append((b, _write_out(i, b)))
    for b, dst in dsts:
        pltpu.make_async_copy(rows_v.at[b], dst, sw).wait()


def kernel(inputs, pos_embed_weights):
    idx = inputs.astype(jnp.int32).reshape(_NW, _N_CHUNKS, _CHUNK)

    out = _gather_rows(idx, pos_embed_weights)
    return out.reshape(_B, _S, _EMBED_DIM), pos_embed_weights


# speedup vs baseline: 2.3538x; 1.0353x over previous
import functools

import jax
import jax.numpy as jnp
from jax import lax
from jax.experimental import pallas as pl
from jax.experimental.pallas import tpu as pltpu
from jax.experimental.pallas import tpu_sc as plsc

_NUM_POS = 8192
_EMBED_DIM = 768
_B = 4
_S = 8192
_TOTAL = _B * _S

_NC = 2
_NS = 16
_NW = _NC * _NS
_ROWS_PER_W = _TOTAL // _NW
_CHUNK = 32
_N_CHUNKS = _ROWS_PER_W // _CHUNK
_NBUF = 4
_N_GROUPS = _N_CHUNKS // _NBUF

_mesh = plsc.VectorSubcoreMesh(core_axis_name="c", subcore_axis_name="s")


@functools.partial(
    pl.kernel,
    mesh=_mesh,
    out_type=jax.ShapeDtypeStruct((_TOTAL, _EMBED_DIM), jnp.float32),
    scratch_types=[
        pltpu.VMEM((_N_CHUNKS, _CHUNK), jnp.int32),
        pltpu.VMEM((_NBUF, _CHUNK, _EMBED_DIM), jnp.float32),
        pltpu.SemaphoreType.DMA,
        pltpu.SemaphoreType.DMA,
    ],
)
def _gather_rows(idx_hbm, table_hbm, out_hbm, idx_v, rows_v, sg, sw):
    wid = lax.axis_index("s") * _NC + lax.axis_index("c")
    base = wid * _ROWS_PER_W
    pltpu.sync_copy(idx_hbm.at[wid], idx_v)

    for b in range(_NBUF):
        pltpu.async_copy(table_hbm.at[idx_v.at[b]], rows_v.at[b], sg)

    def _wait_gather(b):
        pltpu.make_async_copy(
            table_hbm.at[idx_v.at[0]], rows_v.at[b], sg).wait()

    def _write_out(i, b):
        dst = out_hbm.at[pl.ds(base + i * _CHUNK, _CHUNK)]
        pltpu.async_copy(rows_v.at[b], dst, sw)
        return dst

    def _group(g, carry):
        for b in range(_NBUF):
            i = g * _NBUF + b
            _wait_gather(b)
            dst = _write_out(i, b)
            pltpu.make_async_copy(rows_v.at[b], dst, sw).wait()
            pltpu.async_copy(
                table_hbm.at[idx_v.at[i + _NBUF]], rows_v.at[b], sg)
        return carry

    lax.fori_loop(0, _N_GROUPS - 1, _group, 0)

    dsts = []
    for b in range(_NBUF):
        i = (_N_GROUPS - 1) * _NBUF + b
        _wait_gather(b)
        dsts.append((b, _write_out(i, b)))
    for b, dst in dsts:
        pltpu.make_async_copy(rows_v.at[b], dst, sw).wait()


def _copy_body(w_ref, o_ref):
    o_ref[...] = w_ref[...]


def _weights_passthrough(w):
    return pl.pallas_call(
        _copy_body,
        grid=(16,),
        in_specs=[pl.BlockSpec((_NUM_POS // 16, _EMBED_DIM),
                               lambda i: (i, 0))],
        out_specs=pl.BlockSpec((_NUM_POS // 16, _EMBED_DIM),
                               lambda i: (i, 0)),
        out_shape=jax.ShapeDtypeStruct((_NUM_POS, _EMBED_DIM), jnp.float32),
    )(w)


def kernel(inputs, pos_embed_weights):
    idx = inputs.astype(jnp.int32).reshape(_NW, _N_CHUNKS, _CHUNK)

    out = _gather_rows(idx, pos_embed_weights)
    w_out = _weights_passthrough(pos_embed_weights)
    return out.reshape(_B, _S, _EMBED_DIM), w_out
